# baseline (speedup 1.0000x reference)
import jax
import jax.numpy as jnp
from jax import lax
from jax.experimental import pallas as pl
from jax.experimental.pallas import tpu as pltpu
from jax.experimental.pallas import tpu_sc as plsc

B, S, D = 4, 4096, 1024
ND = 2047
NDP = 2048
NC, NS, L = 2, 16, 16
NW = NC * NS
RPB = NW // B
RD = NDP // RPB
NP = 8
CW = D // NP
TCH = 128
NCH = S // TCH
AR = RD + 8
ZB = 64


def _body(x_hbm, dst_hbm, pos_hbm, out_x, out_pos,
          dst_v, pos_v, cnt_v, ps_v, recip_v, pout_v, idx2,
          tok_a, tok_b, eb_a, eb_b, zbuf, acc_sh,
          sem_a, sem_b, sem_z, sem_r, sem_e0, sem_e1, sem_w, sem_s):
    cid = lax.axis_index("c")
    sid = lax.axis_index("s")
    wid = sid * NC + cid
    b = wid // RPB
    r = wid % RPB
    r0 = r * RD
    sbase = sid * AR

    pltpu.sync_copy(dst_hbm.at[pl.ds(pl.multiple_of(b * S, S), S)], dst_v)
    pltpu.sync_copy(pos_hbm.at[pl.ds(pl.multiple_of(b * S, S), S)], pos_v)

    zi = jnp.zeros((L,), jnp.int32)

    def cnt_lt(i, carry):
        a0, a1 = carry
        v = dst_v[pl.ds(i * L, L)]
        a0 = a0 + jnp.where(v < r0, 1, 0)
        a1 = a1 + jnp.where(v < r0 + RD, 1, 0)
        return a0, a1

    a0, a1 = lax.fori_loop(0, S // L, cnt_lt, (zi, zi))
    t0 = jnp.sum(a0)
    t1 = jnp.sum(a1)

    zf = jnp.zeros((L,), jnp.float32)

    def zc(i, _):
        cnt_v[pl.ds(i * L, L)] = zi
        ps_v[pl.ds(i * L, L)] = zi
        return 0

    lax.fori_loop(0, RD // L, zc, 0)

    def zz(row, _):
        for cc in range(CW // L):
            zbuf[row, pl.ds(cc * L, L)] = zf
        return 0

    lax.fori_loop(0, ZB, zz, 0)

    lanes = lax.iota(jnp.int32, L)
    ones = jnp.ones((L,), jnp.int32)
    i0 = lax.div(t0, L)
    i1 = lax.div(t1 + (L - 1), L)

    def cvec(i, _):
        g = i * L
        dv = dst_v[pl.ds(g, L)]
        pv = pos_v[pl.ds(g, L)]
        gi = g + lanes
        msk = (gi >= t0) & (gi < t1)
        d = dv - r0
        plsc.addupdate_scatter(cnt_v, [d], ones, mask=msk)
        plsc.addupdate_scatter(ps_v, [d], pv, mask=msk)
        return 0

    lax.fori_loop(i0, i1, cvec, 0)

    onef = jnp.ones((L,), jnp.float32)

    def rc(i, _):
        sl = pl.ds(i * L, L)
        cv = cnt_v[sl]
        cf = cv.astype(jnp.float32)
        recip_v[sl] = onef / jnp.maximum(cf, 1.0)
        pout_v[sl] = lax.div(ps_v[sl], jnp.maximum(cv, 1))
        return 0

    lax.fori_loop(0, RD // L, rc, 0)

    pltpu.sync_copy(pout_v, out_pos.at[pl.ds(pl.multiple_of(b * NDP + r0, RD), RD)])

    def mkidx(ch, _):
        for k in range(TCH // L):
            dv = dst_v[pl.ds(ch * TCH + k * L, L)] - r0
            ok = (dv >= 0) & (dv < RD)
            idx2[ch, pl.ds(k * L, L)] = sbase + jnp.where(ok, dv, RD)
        return 0

    lax.fori_loop(0, NCH, mkidx, 0)

    ch0 = lax.div(t0, TCH)
    ch1 = lax.div(t1 + (TCH - 1), TCH)
    nch = ch1 - ch0

    def src_for(c, p):
        return x_hbm.at[b, pl.ds(pl.multiple_of(c * TCH, TCH), TCH),
                        pl.ds(pl.multiple_of(p * CW, CW), CW)]

    SH = RD // 2

    def acc_half(h, j):
        return acc_sh.at[
            pl.ds(pl.multiple_of(sbase + h * SH + j * ZB, 8), ZB)]

    def acc_slab(h):
        return acc_sh.at[pl.ds(pl.multiple_of(sbase + h * SH, 8), SH)]

    def out_slab(h, p):
        return out_x.at[b, pl.ds(pl.multiple_of(r0 + h * SH, SH), SH),
                        pl.ds(pl.multiple_of(p * CW, CW), CW)]

    def issue_zeros(p):
        pltpu.async_copy(zbuf, acc_half(0, 0), sem_z)
        pltpu.async_copy(zbuf, acc_half(0, 1), sem_z)
        pltpu.async_copy(zbuf, acc_half(1, 0), sem_r)
        pltpu.async_copy(zbuf, acc_half(1, 1), sem_r)

    def wait_zeros():
        pltpu.make_async_copy(zbuf, acc_half(0, 0), sem_z).wait()
        pltpu.make_async_copy(zbuf, acc_half(0, 1), sem_z).wait()
        pltpu.make_async_copy(zbuf, acc_half(1, 0), sem_r).wait()
        pltpu.make_async_copy(zbuf, acc_half(1, 1), sem_r).wait()

    def scale(buf, h):
        def sgr(gr, _):
            rv = recip_v[pl.ds(h * SH + gr * L, L)]
            for j in range(L):
                rcp = rv[j]
                row = gr * L + j
                for cc in range(CW // L):
                    sl = pl.ds(cc * L, L)
                    buf[row, sl] = buf[row, sl] * rcp
            return 0

        lax.fori_loop(0, SH // L, sgr, 0)

    issue_zeros(0)

    @pl.when(nch > 0)
    def _():
        pltpu.async_copy(src_for(ch0, 0), tok_a, sem_a)

    def one_pass(p, _):
        wait_zeros()

        bufs = ((tok_a, sem_a), (tok_b, sem_b))

        def pair(cp, _):
            for k in range(2):
                buf, sem = bufs[k]
                nbuf, nsem = bufs[1 - k]
                c = ch0 + cp * 2 + k

                @pl.when(c < ch1)
                def _():
                    pltpu.make_async_copy(src_for(c, p), buf, sem).wait()

                    @pl.when(c > ch0)
                    def _():
                        pltpu.make_async_copy(
                            nbuf, acc_sh.at[idx2.at[c - 1]], sem_s).wait()

                    @pl.when(c + 1 < ch1)
                    def _():
                        pltpu.async_copy(src_for(c + 1, p), nbuf, nsem)

                    pltpu.async_copy(
                        buf, acc_sh.at[idx2.at[c]], sem_s, add=True)

            return 0

        lax.fori_loop(0, lax.div(nch + 1, 2), pair, 0)

        @pl.when(nch > 0)
        def _():
            lastc = ch1 - 1
            for k in range(2):

                @pl.when((lastc - ch0) % 2 == k)
                def _(k=k):
                    pltpu.make_async_copy(
                        bufs[k][0], acc_sh.at[idx2.at[lastc]], sem_s).wait()

        @pl.when(p > 0)
        def _():
            pltpu.make_async_copy(eb_a, out_slab(0, p - 1), sem_w).wait()
            pltpu.make_async_copy(eb_b, out_slab(1, p - 1), sem_w).wait()

        pltpu.async_copy(acc_slab(0), eb_a, sem_e0)
        pltpu.async_copy(acc_slab(1), eb_b, sem_e1)

        pltpu.make_async_copy(acc_slab(0), eb_a, sem_e0).wait()
        scale(eb_a, 0)
        pltpu.async_copy(eb_a, out_slab(0, p), sem_w)

        pltpu.make_async_copy(acc_slab(1), eb_b, sem_e1).wait()

        @pl.when(p + 1 < NP)
        def _():
            issue_zeros(p + 1)

            @pl.when(nch > 0)
            def _():
                pltpu.async_copy(src_for(ch0, p + 1), tok_a, sem_a)

        scale(eb_b, 1)
        pltpu.async_copy(eb_b, out_slab(1, p), sem_w)
        return 0

    lax.fori_loop(0, NP, one_pass, 0)
    pltpu.make_async_copy(eb_a, out_slab(0, NP - 1), sem_w).wait()
    pltpu.make_async_copy(eb_b, out_slab(1, NP - 1), sem_w).wait()


@jax.jit
def _downsample(x, dst, pos):
    mesh = plsc.VectorSubcoreMesh(
        core_axis_name="c", subcore_axis_name="s",
        num_cores=NC, num_subcores=NS)
    f = pl.kernel(
        _body,
        out_type=(jax.ShapeDtypeStruct((B, NDP, D), jnp.float32),
                  jax.ShapeDtypeStruct((B * NDP,), jnp.int32)),
        mesh=mesh,
        compiler_params=pltpu.CompilerParams(
            use_tc_tiling_on_sc=True, needs_layout_passes=False),
        scratch_types=[
            pltpu.VMEM((S,), jnp.int32),
            pltpu.VMEM((S,), jnp.int32),
            pltpu.VMEM((RD,), jnp.int32),
            pltpu.VMEM((RD,), jnp.int32),
            pltpu.VMEM((RD,), jnp.float32),
            pltpu.VMEM((RD,), jnp.int32),
            pltpu.VMEM((NCH, TCH), jnp.int32),
            pltpu.VMEM((TCH, CW), jnp.float32),
            pltpu.VMEM((TCH, CW), jnp.float32),
            pltpu.VMEM((RD // 2, CW), jnp.float32),
            pltpu.VMEM((RD // 2, CW), jnp.float32),
            pltpu.VMEM((ZB, CW), jnp.float32),
            pltpu.VMEM_SHARED((NS * AR, CW), jnp.float32),
            pltpu.SemaphoreType.DMA,
            pltpu.SemaphoreType.DMA,
            pltpu.SemaphoreType.DMA,
            pltpu.SemaphoreType.DMA,
            pltpu.SemaphoreType.DMA,
            pltpu.SemaphoreType.DMA,
            pltpu.SemaphoreType.DMA,
            pltpu.SemaphoreType.DMA,
        ],
    )
    return f(x, dst, pos)


def kernel(x, position_ids, down_merge_dst, n_dst):
    xo, po = _downsample(
        x, down_merge_dst.reshape(-1), position_ids.reshape(-1))
    return (xo[:, :ND], po.reshape(B, NDP)[:, :ND])


# speedup vs baseline: 1.1447x; 1.1447x over previous
import jax
import jax.numpy as jnp
from jax import lax
from jax.experimental import pallas as pl
from jax.experimental.pallas import tpu as pltpu
from jax.experimental.pallas import tpu_sc as plsc

B, S, D = 4, 4096, 1024
ND = 2047
NDP = 2048
NC, NS, L = 2, 16, 16
NW = NC * NS
RPB = NW // B
RD = NDP // RPB
NP = 8
CW = D // NP
TCH = 128
NCH = S // TCH
AR = RD + 8
ZB = 64


def _body(x_hbm, dst_hbm, pos_hbm, out_x, out_pos,
          dst_v, pos_v, cnt_v, ps_v, recip_v, pout_v, idx2,
          tok_a, tok_b, zbuf, acc_sh,
          sem_a, sem_b, sem_z, sem_r, sem_w, sem_s):
    cid = lax.axis_index("c")
    sid = lax.axis_index("s")
    wid = sid * NC + cid
    b = wid // RPB
    r = wid % RPB
    r0 = r * RD
    sbase = sid * AR

    pltpu.sync_copy(dst_hbm.at[pl.ds(pl.multiple_of(b * S, S), S)], dst_v)
    pltpu.sync_copy(pos_hbm.at[pl.ds(pl.multiple_of(b * S, S), S)], pos_v)

    zi = jnp.zeros((L,), jnp.int32)

    def cnt_lt(i, carry):
        a0, a1 = carry
        v = dst_v[pl.ds(i * L, L)]
        a0 = a0 + jnp.where(v < r0, 1, 0)
        a1 = a1 + jnp.where(v < r0 + RD, 1, 0)
        return a0, a1

    a0, a1 = lax.fori_loop(0, S // L, cnt_lt, (zi, zi))
    t0 = jnp.sum(a0)
    t1 = jnp.sum(a1)

    zf = jnp.zeros((L,), jnp.float32)

    def zc(i, _):
        cnt_v[pl.ds(i * L, L)] = zi
        ps_v[pl.ds(i * L, L)] = zi
        return 0

    lax.fori_loop(0, RD // L, zc, 0)

    def zz(row, _):
        for cc in range(CW // L):
            zbuf[row, pl.ds(cc * L, L)] = zf
        return 0

    lax.fori_loop(0, ZB, zz, 0)

    lanes = lax.iota(jnp.int32, L)
    ones = jnp.ones((L,), jnp.int32)
    i0 = lax.div(t0, L)
    i1 = lax.div(t1 + (L - 1), L)

    def cvec(i, _):
        g = i * L
        dv = dst_v[pl.ds(g, L)]
        pv = pos_v[pl.ds(g, L)]
        gi = g + lanes
        msk = (gi >= t0) & (gi < t1)
        d = dv - r0
        plsc.addupdate_scatter(cnt_v, [d], ones, mask=msk)
        plsc.addupdate_scatter(ps_v, [d], pv, mask=msk)
        return 0

    lax.fori_loop(i0, i1, cvec, 0)

    onef = jnp.ones((L,), jnp.float32)

    def rc(i, _):
        sl = pl.ds(i * L, L)
        cv = cnt_v[sl]
        cf = cv.astype(jnp.float32)
        recip_v[sl] = onef / jnp.maximum(cf, 1.0)
        pout_v[sl] = lax.div(ps_v[sl], jnp.maximum(cv, 1))
        return 0

    lax.fori_loop(0, RD // L, rc, 0)

    pltpu.sync_copy(pout_v, out_pos.at[pl.ds(pl.multiple_of(b * NDP + r0, RD), RD)])

    def mkidx(ch, _):
        for k in range(TCH // L):
            dv = dst_v[pl.ds(ch * TCH + k * L, L)] - r0
            ok = (dv >= 0) & (dv < RD)
            idx2[ch, pl.ds(k * L, L)] = sbase + jnp.where(ok, dv, RD)
        return 0

    lax.fori_loop(0, NCH, mkidx, 0)

    ch0 = lax.div(t0, TCH)
    ch1 = lax.div(t1 + (TCH - 1), TCH)
    nch = ch1 - ch0

    def src_for(c, p):
        return x_hbm.at[b, pl.ds(pl.multiple_of(c * TCH, TCH), TCH),
                        pl.ds(pl.multiple_of(p * CW, CW), CW)]

    SH = RD // 2

    def acc_half(h, j):
        return acc_sh.at[
            pl.ds(pl.multiple_of(sbase + h * SH + j * ZB, 8), ZB)]

    def acc_slab(h):
        return acc_sh.at[pl.ds(pl.multiple_of(sbase + h * SH, 8), SH)]

    def out_slab(h, p):
        return out_x.at[b, pl.ds(pl.multiple_of(r0 + h * SH, SH), SH),
                        pl.ds(pl.multiple_of(p * CW, CW), CW)]

    def issue_zeros(p):
        pltpu.async_copy(zbuf, acc_half(0, 0), sem_z)
        pltpu.async_copy(zbuf, acc_half(0, 1), sem_z)
        pltpu.async_copy(zbuf, acc_half(1, 0), sem_r)
        pltpu.async_copy(zbuf, acc_half(1, 1), sem_r)

    def wait_zeros():
        pltpu.make_async_copy(zbuf, acc_half(0, 0), sem_z).wait()
        pltpu.make_async_copy(zbuf, acc_half(0, 1), sem_z).wait()
        pltpu.make_async_copy(zbuf, acc_half(1, 0), sem_r).wait()
        pltpu.make_async_copy(zbuf, acc_half(1, 1), sem_r).wait()

    issue_zeros(0)

    @pl.when(nch > 0)
    def _():
        pltpu.async_copy(src_for(ch0, 0), tok_a, sem_a)

    def one_pass(p, _):
        wait_zeros()

        bufs = ((tok_a, sem_a), (tok_b, sem_b))

        def pair(cp, _):
            for k in range(2):
                buf, sem = bufs[k]
                nbuf, nsem = bufs[1 - k]
                c = ch0 + cp * 2 + k

                @pl.when(c < ch1)
                def _():
                    pltpu.make_async_copy(src_for(c, p), buf, sem).wait()

                    @pl.when(c > ch0)
                    def _():
                        pltpu.make_async_copy(
                            nbuf, acc_sh.at[idx2.at[c - 1]], sem_s).wait()

                    @pl.when(c + 1 < ch1)
                    def _():
                        pltpu.async_copy(src_for(c + 1, p), nbuf, nsem)

                    def presc(gr, _):
                        dv = dst_v[pl.ds(c * TCH + gr * L, L)] - r0
                        dvc = jnp.minimum(jnp.maximum(dv, 0), RD - 1)
                        rvec = plsc.load_gather(recip_v, [dvc])
                        for j in range(L):
                            rcp = rvec[j]
                            row = gr * L + j
                            for cc in range(CW // L):
                                sl = pl.ds(cc * L, L)
                                buf[row, sl] = buf[row, sl] * rcp
                        return 0

                    lax.fori_loop(0, TCH // L, presc, 0)

                    pltpu.async_copy(
                        buf, acc_sh.at[idx2.at[c]], sem_s, add=True)

            return 0

        lax.fori_loop(0, lax.div(nch + 1, 2), pair, 0)

        @pl.when(nch > 0)
        def _():
            lastc = ch1 - 1
            for k in range(2):

                @pl.when((lastc - ch0) % 2 == k)
                def _(k=k):
                    pltpu.make_async_copy(
                        bufs[k][0], acc_sh.at[idx2.at[lastc]], sem_s).wait()

        pltpu.async_copy(acc_slab(0), out_slab(0, p), sem_w)
        pltpu.async_copy(acc_slab(1), out_slab(1, p), sem_w)
        pltpu.make_async_copy(acc_slab(0), out_slab(0, p), sem_w).wait()
        pltpu.make_async_copy(acc_slab(1), out_slab(1, p), sem_w).wait()

        @pl.when(p + 1 < NP)
        def _():
            issue_zeros(p + 1)

            @pl.when(nch > 0)
            def _():
                pltpu.async_copy(src_for(ch0, p + 1), tok_a, sem_a)

        return 0

    lax.fori_loop(0, NP, one_pass, 0)


@jax.jit
def _downsample(x, dst, pos):
    mesh = plsc.VectorSubcoreMesh(
        core_axis_name="c", subcore_axis_name="s",
        num_cores=NC, num_subcores=NS)
    f = pl.kernel(
        _body,
        out_type=(jax.ShapeDtypeStruct((B, NDP, D), jnp.float32),
                  jax.ShapeDtypeStruct((B * NDP,), jnp.int32)),
        mesh=mesh,
        compiler_params=pltpu.CompilerParams(
            use_tc_tiling_on_sc=True, needs_layout_passes=False),
        scratch_types=[
            pltpu.VMEM((S,), jnp.int32),
            pltpu.VMEM((S,), jnp.int32),
            pltpu.VMEM((RD,), jnp.int32),
            pltpu.VMEM((RD,), jnp.int32),
            pltpu.VMEM((RD,), jnp.float32),
            pltpu.VMEM((RD,), jnp.int32),
            pltpu.VMEM((NCH, TCH), jnp.int32),
            pltpu.VMEM((TCH, CW), jnp.float32),
            pltpu.VMEM((TCH, CW), jnp.float32),
            pltpu.VMEM((ZB, CW), jnp.float32),
            pltpu.VMEM_SHARED((NS * AR, CW), jnp.float32),
            pltpu.SemaphoreType.DMA,
            pltpu.SemaphoreType.DMA,
            pltpu.SemaphoreType.DMA,
            pltpu.SemaphoreType.DMA,
            pltpu.SemaphoreType.DMA,
            pltpu.SemaphoreType.DMA,
        ],
    )
    return f(x, dst, pos)


def kernel(x, position_ids, down_merge_dst, n_dst):
    xo, po = _downsample(
        x, down_merge_dst.reshape(-1), position_ids.reshape(-1))
    return (xo[:, :ND], po.reshape(B, NDP)[:, :ND])


# speedup vs baseline: 1.1506x; 1.0052x over previous
import jax
import jax.numpy as jnp
from jax import lax
from jax.experimental import pallas as pl
from jax.experimental.pallas import tpu as pltpu
from jax.experimental.pallas import tpu_sc as plsc

B, S, D = 4, 4096, 1024
ND = 2047
NDP = 2048
NC, NS, L = 2, 16, 16
NW = NC * NS
RPB = NW // B
RD = NDP // RPB
NP = 8
CW = D // NP
TCH = 128
NCH = S // TCH
AR = RD + 8
ZB = 64


def _body(x_hbm, dst_hbm, pos_hbm, out_x, out_pos,
          dst_v, pos_v, cnt_v, ps_v, recip_v, pout_v, idx2,
          idx3, tok_a, tok_b, zbuf, acc_sh,
          sem_a, sem_b, sem_z, sem_r, sem_w, sem_w2, sem_s):
    cid = lax.axis_index("c")
    sid = lax.axis_index("s")
    wid = sid * NC + cid
    b = wid // RPB
    r = wid % RPB
    r0 = r * RD
    sbase = sid * AR

    pltpu.sync_copy(dst_hbm.at[pl.ds(pl.multiple_of(b * S, S), S)], dst_v)
    pltpu.sync_copy(pos_hbm.at[pl.ds(pl.multiple_of(b * S, S), S)], pos_v)

    zi = jnp.zeros((L,), jnp.int32)

    def cnt_lt(i, carry):
        a0, a1 = carry
        v = dst_v[pl.ds(i * L, L)]
        a0 = a0 + jnp.where(v < r0, 1, 0)
        a1 = a1 + jnp.where(v < r0 + RD, 1, 0)
        return a0, a1

    a0, a1 = lax.fori_loop(0, S // L, cnt_lt, (zi, zi))
    t0 = jnp.sum(a0)
    t1 = jnp.sum(a1)

    zf = jnp.zeros((L,), jnp.float32)

    def zc(i, _):
        cnt_v[pl.ds(i * L, L)] = zi
        ps_v[pl.ds(i * L, L)] = zi
        return 0

    lax.fori_loop(0, RD // L, zc, 0)

    def zz(row, _):
        for cc in range(CW // L):
            zbuf[row, pl.ds(cc * L, L)] = zf
        return 0

    lax.fori_loop(0, ZB, zz, 0)

    lanes = lax.iota(jnp.int32, L)
    ones = jnp.ones((L,), jnp.int32)
    i0 = lax.div(t0, L)
    i1 = lax.div(t1 + (L - 1), L)

    def cvec(i, _):
        g = i * L
        dv = dst_v[pl.ds(g, L)]
        pv = pos_v[pl.ds(g, L)]
        gi = g + lanes
        msk = (gi >= t0) & (gi < t1)
        d = dv - r0
        plsc.addupdate_scatter(cnt_v, [d], ones, mask=msk)
        plsc.addupdate_scatter(ps_v, [d], pv, mask=msk)
        return 0

    lax.fori_loop(i0, i1, cvec, 0)

    onef = jnp.ones((L,), jnp.float32)

    def rc(i, _):
        sl = pl.ds(i * L, L)
        cv = cnt_v[sl]
        cf = cv.astype(jnp.float32)
        recip_v[sl] = onef / jnp.maximum(cf, 1.0)
        pout_v[sl] = lax.div(ps_v[sl], jnp.maximum(cv, 1))
        return 0

    lax.fori_loop(0, RD // L, rc, 0)

    pltpu.sync_copy(pout_v, out_pos.at[pl.ds(pl.multiple_of(b * NDP + r0, RD), RD)])

    NSAR = NS * AR

    def mkidx(ch, _):
        for k in range(TCH // L):
            dv = dst_v[pl.ds(ch * TCH + k * L, L)] - r0
            ok = (dv >= 0) & (dv < RD)
            v = sbase + jnp.where(ok, dv, RD)
            idx2[ch, pl.ds(k * L, L)] = v
            idx3[ch, pl.ds(k * L, L)] = v + NSAR
        return 0

    lax.fori_loop(0, NCH, mkidx, 0)

    ch0 = lax.div(t0, TCH)
    ch1 = lax.div(t1 + (TCH - 1), TCH)
    nch = ch1 - ch0

    def src_for(c, p):
        return x_hbm.at[b, pl.ds(pl.multiple_of(c * TCH, TCH), TCH),
                        pl.ds(pl.multiple_of(p * CW, CW), CW)]

    SH = RD // 2

    def acc_half(bk, h, j):
        return acc_sh.at[pl.ds(
            pl.multiple_of(bk * NSAR + sbase + h * SH + j * ZB, 8), ZB)]

    def acc_slab(bk, h):
        return acc_sh.at[pl.ds(
            pl.multiple_of(bk * NSAR + sbase + h * SH, 8), SH)]

    def out_slab(h, p):
        return out_x.at[b, pl.ds(pl.multiple_of(r0 + h * SH, SH), SH),
                        pl.ds(pl.multiple_of(p * CW, CW), CW)]

    zsems = (sem_z, sem_r)
    wsems = (sem_w, sem_w2)

    def issue_zeros(bk):
        for h in range(2):
            for j in range(2):
                pltpu.async_copy(zbuf, acc_half(bk, h, j), zsems[bk])

    def wait_zeros(bk):
        for h in range(2):
            for j in range(2):
                pltpu.make_async_copy(zbuf, acc_half(bk, h, j),
                                      zsems[bk]).wait()

    issue_zeros(0)
    issue_zeros(1)

    @pl.when(nch > 0)
    def _():
        pltpu.async_copy(src_for(ch0, 0), tok_a, sem_a)

    def pass_pair(pp, _):
        for bk in range(2):
            p = pp * 2 + bk
            idxk = (idx2, idx3)[bk]

            wait_zeros(bk)

            bufs = ((tok_a, sem_a), (tok_b, sem_b))

            def pair(cp, _, p=p, idxk=idxk, bufs=bufs):
                for k in range(2):
                    buf, sem = bufs[k]
                    nbuf, nsem = bufs[1 - k]
                    c = ch0 + cp * 2 + k

                    @pl.when(c < ch1)
                    def _():
                        pltpu.make_async_copy(src_for(c, p), buf, sem).wait()

                        @pl.when(c > ch0)
                        def _():
                            pltpu.make_async_copy(
                                nbuf, acc_sh.at[idxk.at[c - 1]],
                                sem_s).wait()

                        @pl.when(c + 1 < ch1)
                        def _():
                            pltpu.async_copy(src_for(c + 1, p), nbuf, nsem)

                        def presc(gr, _):
                            dv = dst_v[pl.ds(c * TCH + gr * L, L)] - r0
                            dvc = jnp.minimum(jnp.maximum(dv, 0), RD - 1)
                            rvec = plsc.load_gather(recip_v, [dvc])
                            for j in range(L):
                                rcp = rvec[j]
                                row = gr * L + j
                                for cc in range(CW // L):
                                    sl = pl.ds(cc * L, L)
                                    buf[row, sl] = buf[row, sl] * rcp
                            return 0

                        lax.fori_loop(0, TCH // L, presc, 0)

                        pltpu.async_copy(
                            buf, acc_sh.at[idxk.at[c]], sem_s,
                            add=True)

                return 0

            lax.fori_loop(0, lax.div(nch + 1, 2), pair, 0)

            @pl.when(nch > 0)
            def _(idxk=idxk, bufs=bufs):
                lastc = ch1 - 1
                for k in range(2):

                    @pl.when((lastc - ch0) % 2 == k)
                    def _(k=k):
                        pltpu.make_async_copy(
                            bufs[k][0], acc_sh.at[idxk.at[lastc]],
                            sem_s).wait()

            pltpu.async_copy(acc_slab(bk, 0), out_slab(0, p), wsems[bk])
            pltpu.async_copy(acc_slab(bk, 1), out_slab(1, p), wsems[bk])

            @pl.when(p + 1 < NP)
            def _(p=p, bk=bk):
                @pl.when(nch > 0)
                def _():
                    pltpu.async_copy(src_for(ch0, p + 1), tok_a, sem_a)

                @pl.when(p >= 1)
                def _():
                    ob = 1 - bk
                    pltpu.make_async_copy(
                        acc_slab(ob, 0), out_slab(0, p - 1),
                        wsems[ob]).wait()
                    pltpu.make_async_copy(
                        acc_slab(ob, 1), out_slab(1, p - 1),
                        wsems[ob]).wait()
                    issue_zeros(ob)

        return 0

    lax.fori_loop(0, NP // 2, pass_pair, 0)
    for bk, p in ((0, NP - 2), (1, NP - 1)):
        pltpu.make_async_copy(acc_slab(bk, 0), out_slab(0, p),
                              wsems[bk]).wait()
        pltpu.make_async_copy(acc_slab(bk, 1), out_slab(1, p),
                              wsems[bk]).wait()


@jax.jit
def _downsample(x, dst, pos):
    mesh = plsc.VectorSubcoreMesh(
        core_axis_name="c", subcore_axis_name="s",
        num_cores=NC, num_subcores=NS)
    f = pl.kernel(
        _body,
        out_type=(jax.ShapeDtypeStruct((B, NDP, D), jnp.float32),
                  jax.ShapeDtypeStruct((B * NDP,), jnp.int32)),
        mesh=mesh,
        compiler_params=pltpu.CompilerParams(
            use_tc_tiling_on_sc=True, needs_layout_passes=False),
        scratch_types=[
            pltpu.VMEM((S,), jnp.int32),
            pltpu.VMEM((S,), jnp.int32),
            pltpu.VMEM((RD,), jnp.int32),
            pltpu.VMEM((RD,), jnp.int32),
            pltpu.VMEM((RD,), jnp.float32),
            pltpu.VMEM((RD,), jnp.int32),
            pltpu.VMEM((NCH, TCH), jnp.int32),
            pltpu.VMEM((NCH, TCH), jnp.int32),
            pltpu.VMEM((TCH, CW), jnp.float32),
            pltpu.VMEM((TCH, CW), jnp.float32),
            pltpu.VMEM((ZB, CW), jnp.float32),
            pltpu.VMEM_SHARED((2 * NS * AR, CW), jnp.float32),
            pltpu.SemaphoreType.DMA,
            pltpu.SemaphoreType.DMA,
            pltpu.SemaphoreType.DMA,
            pltpu.SemaphoreType.DMA,
            pltpu.SemaphoreType.DMA,
            pltpu.SemaphoreType.DMA,
            pltpu.SemaphoreType.DMA,
        ],
    )
    return f(x, dst, pos)


def kernel(x, position_ids, down_merge_dst, n_dst):
    xo, po = _downsample(
        x, down_merge_dst.reshape(-1), position_ids.reshape(-1))
    return (xo[:, :ND], po.reshape(B, NDP)[:, :ND])
